# baseline (speedup 1.0000x reference)
import numpy as np
import jax
import jax.numpy as jnp
from jax import lax
from jax.experimental import pallas as pl
from jax.experimental.pallas import tpu as pltpu
from jax.experimental.pallas import tpu_sc as plsc

_VOCAB = 100000
_D = 768
_SEQ = 2048
_BATCH = 4
_L = 16
_NVEC = _D // _L

_NC, _NS = 2, 16
_NW = _NC * _NS
_ROWS = _SEQ * _BATCH
_RPW = _ROWS // _NW
_PPW = _RPW // _BATCH
_CHUNK = 64
_NCHUNK = _RPW // _CHUNK
_PPC = _CHUNK // _BATCH


def _pe_table() -> np.ndarray:
    position = np.arange(0.0, _SEQ)[:, None]
    div_term = np.exp(np.arange(0.0, _D, 2) * -(np.log(10000.0) / _D))
    pe = np.zeros((_SEQ, _D), dtype=np.float32)
    pe[:, 0::2] = np.sin(position * div_term)
    pe[:, 1::2] = np.cos(position * div_term)
    return pe


_PE = _pe_table()

_mesh = plsc.VectorSubcoreMesh(core_axis_name="c", subcore_axis_name="s")


def _emb_lookup(idx, table, pe):
    @pl.kernel(
        out_type=jax.ShapeDtypeStruct((_ROWS, _D), jnp.float32),
        mesh=_mesh,
        scratch_types=[
            pltpu.VMEM((_RPW,), jnp.int32),
            pltpu.VMEM((_PPW, _D), jnp.float32),
            pltpu.VMEM((_CHUNK, _D), jnp.float32),
            pltpu.SemaphoreType.DMA,
        ],
    )
    def body(idx_hbm, table_hbm, pe_hbm, out_hbm, idx_v, pe_v, buf_v, sem):
        wid = lax.axis_index("s") * _NC + lax.axis_index("c")
        base = wid * _RPW
        pbase = wid * _PPW
        pltpu.sync_copy(idx_hbm.at[pl.ds(base, _RPW)], idx_v)
        pltpu.sync_copy(pe_hbm.at[pl.ds(pbase, _PPW)], pe_v)
        for c in range(_NCHUNK):
            off = base + c * _CHUNK
            pltpu.async_copy(
                table_hbm.at[idx_v.at[pl.ds(c * _CHUNK, _CHUNK)]],
                buf_v,
                sem,
            ).wait()

            def add_pe(p, _, c=c):
                pos = c * _PPC + p
                for j in range(_NVEC):
                    v = pe_v[pos, pl.ds(j * _L, _L)]
                    for r in range(_BATCH):
                        plsc.addupdate(buf_v.at[p * _BATCH + r, pl.ds(j * _L, _L)], v)
                return 0

            lax.fori_loop(0, _PPC, add_pe, 0)
            pltpu.sync_copy(buf_v, out_hbm.at[pl.ds(off, _CHUNK)])

    return body(idx, table, pe)


def kernel(x, W):
    idx = x.reshape(_ROWS)
    pe = jnp.asarray(_PE)
    out = _emb_lookup(idx, W, pe)
    return out.reshape(_SEQ, _BATCH, _D)


# speedup vs baseline: 1.1168x; 1.1168x over previous
import numpy as np
import jax
import jax.numpy as jnp
from jax import lax
from jax.experimental import pallas as pl
from jax.experimental.pallas import tpu as pltpu
from jax.experimental.pallas import tpu_sc as plsc

_VOCAB = 100000
_D = 768
_SEQ = 2048
_BATCH = 4
_L = 16
_NVEC = _D // _L

_NC, _NS = 2, 16
_NW = _NC * _NS
_ROWS = _SEQ * _BATCH
_RPW = _ROWS // _NW
_CHUNK = 64
_NCHUNK = _RPW // _CHUNK
_PPC = _CHUNK // _BATCH


def _pe_table() -> np.ndarray:
    position = np.arange(0.0, _SEQ)[:, None]
    div_term = np.exp(np.arange(0.0, _D, 2) * -(np.log(10000.0) / _D))
    pe = np.zeros((_SEQ, _D), dtype=np.float32)
    pe[:, 0::2] = np.sin(position * div_term)
    pe[:, 1::2] = np.cos(position * div_term)
    return pe


_PE = _pe_table()

_mesh = plsc.VectorSubcoreMesh(core_axis_name="c", subcore_axis_name="s")


def _emb_lookup(idx, table, pe):
    @pl.kernel(
        out_type=jax.ShapeDtypeStruct((_ROWS, _D), jnp.float32),
        mesh=_mesh,
        scratch_types=[
            pltpu.VMEM((_RPW,), jnp.int32),
            pltpu.VMEM((_PPC, _D), jnp.float32),
            pltpu.VMEM((_PPC, _D), jnp.float32),
            pltpu.VMEM((_CHUNK, _D), jnp.float32),
            pltpu.VMEM((_CHUNK, _D), jnp.float32),
            pltpu.SemaphoreType.DMA((2,)),
            pltpu.SemaphoreType.DMA((2,)),
            pltpu.SemaphoreType.DMA((2,)),
        ],
    )
    def body(idx_hbm, table_hbm, pe_hbm, out_hbm, idx_v, pe_v0, pe_v1,
             buf_v0, buf_v1, sem_g, sem_p, sem_w):
        pe_v = [pe_v0, pe_v1]
        buf_v = [buf_v0, buf_v1]
        wid = lax.axis_index("s") * _NC + lax.axis_index("c")
        base = wid * _RPW
        pbase = wid * (_RPW // _BATCH)
        pltpu.sync_copy(idx_hbm.at[pl.ds(base, _RPW)], idx_v)

        def start_in(c):
            b = c % 2
            g = pltpu.async_copy(
                table_hbm.at[idx_v.at[pl.ds(c * _CHUNK, _CHUNK)]],
                buf_v[b], sem_g.at[b])
            p = pltpu.async_copy(
                pe_hbm.at[pl.ds(pbase + c * _PPC, _PPC)],
                pe_v[b], sem_p.at[b])
            return g, p

        pending_in = start_in(0)
        pending_out = [None, None]
        for c in range(_NCHUNK):
            b = c % 2
            if c + 1 < _NCHUNK:
                if pending_out[1 - b] is not None:
                    pending_out[1 - b].wait()
                    pending_out[1 - b] = None
                nxt = start_in(c + 1)
            g, p = pending_in
            g.wait()
            p.wait()
            if c + 1 < _NCHUNK:
                pending_in = nxt

            def add_pe(pos, _):
                for j in range(_NVEC):
                    v = pe_v[b][pos, pl.ds(j * _L, _L)]
                    for r in range(_BATCH):
                        plsc.addupdate(
                            buf_v[b].at[pos * _BATCH + r, pl.ds(j * _L, _L)], v)
                return 0

            lax.fori_loop(0, _PPC, add_pe, 0)
            pending_out[b] = pltpu.async_copy(
                buf_v[b], out_hbm.at[pl.ds(base + c * _CHUNK, _CHUNK)],
                sem_w.at[b])
        for w in pending_out:
            if w is not None:
                w.wait()

    return body(idx, table, pe)


def kernel(x, W):
    idx = x.reshape(_ROWS)
    pe = jnp.asarray(_PE)
    out = _emb_lookup(idx, W, pe)
    return out.reshape(_SEQ, _BATCH, _D)


# speedup vs baseline: 1.1212x; 1.0039x over previous
import numpy as np
import jax
import jax.numpy as jnp
from jax import lax
from jax.experimental import pallas as pl
from jax.experimental.pallas import tpu as pltpu
from jax.experimental.pallas import tpu_sc as plsc

_VOCAB = 100000
_D = 768
_SEQ = 2048
_BATCH = 4
_L = 16
_NVEC = _D // _L

_NC, _NS = 2, 16
_NW = _NC * _NS
_ROWS = _SEQ * _BATCH
_RPW = _ROWS // _NW
_CHUNK = 64
_NCHUNK = _RPW // _CHUNK
_PPC = _CHUNK // _BATCH


def _pe_table() -> np.ndarray:
    position = np.arange(0.0, _SEQ)[:, None]
    div_term = np.exp(np.arange(0.0, _D, 2) * -(np.log(10000.0) / _D))
    pe = np.zeros((_SEQ, _D), dtype=np.float32)
    pe[:, 0::2] = np.sin(position * div_term)
    pe[:, 1::2] = np.cos(position * div_term)
    return pe.reshape(_SEQ * _D // 128, 128)


_PE = _pe_table()

_mesh = plsc.VectorSubcoreMesh(core_axis_name="c", subcore_axis_name="s")


def _emb_lookup(idx, table, pe):
    @pl.kernel(
        out_type=jax.ShapeDtypeStruct((_ROWS, _D), jnp.float32),
        mesh=_mesh,
        scratch_types=[
            pltpu.VMEM((_RPW,), jnp.int32),
            pltpu.VMEM((_PPC * 6, 128), jnp.float32),
            pltpu.VMEM((_PPC * 6, 128), jnp.float32),
            pltpu.VMEM((_CHUNK, _D), jnp.float32),
            pltpu.VMEM((_CHUNK, _D), jnp.float32),
            pltpu.SemaphoreType.DMA((2,)),
            pltpu.SemaphoreType.DMA((2,)),
            pltpu.SemaphoreType.DMA((2,)),
        ],
    )
    def body(idx_hbm, table_hbm, pe_hbm, out_hbm, idx_v, pe_v0, pe_v1,
             buf_v0, buf_v1, sem_g, sem_p, sem_w):
        pe_v = [pe_v0, pe_v1]
        buf_v = [buf_v0, buf_v1]
        wid = lax.axis_index("s") * _NC + lax.axis_index("c")
        base = wid * _RPW
        pbase = wid * (_RPW // _BATCH)
        pltpu.sync_copy(idx_hbm.at[pl.ds(base, _RPW)], idx_v)

        def start_in(c):
            b = c % 2
            g = pltpu.async_copy(
                table_hbm.at[idx_v.at[pl.ds(c * _CHUNK, _CHUNK)]],
                buf_v[b], sem_g.at[b])
            p = pltpu.async_copy(
                pe_hbm.at[pl.ds((pbase + c * _PPC) * 6, _PPC * 6)],
                pe_v[b], sem_p.at[b])
            return g, p

        pending_in = start_in(0)
        pending_out = [None, None]
        for c in range(_NCHUNK):
            b = c % 2
            if c + 1 < _NCHUNK:
                if pending_out[1 - b] is not None:
                    pending_out[1 - b].wait()
                    pending_out[1 - b] = None
                nxt = start_in(c + 1)
            g, p = pending_in
            g.wait()
            p.wait()
            if c + 1 < _NCHUNK:
                pending_in = nxt

            def add_pe(pos, _):
                for j in range(_NVEC):
                    v = pe_v[b][pos * 6 + j // 8, pl.ds((j % 8) * _L, _L)]
                    for r in range(_BATCH):
                        plsc.addupdate(
                            buf_v[b].at[pos * _BATCH + r, pl.ds(j * _L, _L)], v)
                return 0

            lax.fori_loop(0, _PPC, add_pe, 0)
            pending_out[b] = pltpu.async_copy(
                buf_v[b], out_hbm.at[pl.ds(base + c * _CHUNK, _CHUNK)],
                sem_w.at[b])
        for w in pending_out:
            if w is not None:
                w.wait()

    return body(idx, table, pe)


def kernel(x, W):
    idx = x.reshape(_ROWS)
    pe = jnp.asarray(_PE)
    out = _emb_lookup(idx, W, pe)
    return out.reshape(_SEQ, _BATCH, _D)


# speedup vs baseline: 1.2407x; 1.1067x over previous
import numpy as np
import jax
import jax.numpy as jnp
from jax import lax
from jax.experimental import pallas as pl
from jax.experimental.pallas import tpu as pltpu
from jax.experimental.pallas import tpu_sc as plsc

_VOCAB = 100000
_D = 768
_SEQ = 2048
_BATCH = 4

_NC, _NS = 2, 16
_NW = _NC * _NS
_PPW = _SEQ // _NW
_CHUNK = _PPW


def _pe_table() -> np.ndarray:
    position = np.arange(0.0, _SEQ)[:, None]
    div_term = np.exp(np.arange(0.0, _D, 2) * -(np.log(10000.0) / _D))
    pe = np.zeros((_SEQ, _D), dtype=np.float32)
    pe[:, 0::2] = np.sin(position * div_term)
    pe[:, 1::2] = np.cos(position * div_term)
    return pe[:, None, :]


_PE = _pe_table()

_mesh = plsc.VectorSubcoreMesh(core_axis_name="c", subcore_axis_name="s")


def _gather(idx_flat, table):
    @pl.kernel(
        out_type=jax.ShapeDtypeStruct((_BATCH, _SEQ, _D), jnp.float32),
        mesh=_mesh,
        scratch_types=[
            pltpu.VMEM((_BATCH * _PPW,), jnp.int32),
            pltpu.VMEM((_CHUNK, _D), jnp.float32),
            pltpu.VMEM((_CHUNK, _D), jnp.float32),
            pltpu.SemaphoreType.DMA((_BATCH,)),
            pltpu.SemaphoreType.DMA((2,)),
            pltpu.SemaphoreType.DMA((2,)),
        ],
    )
    def body(idx_hbm, table_hbm, out_hbm, idx_v, buf_v0, buf_v1,
             sem_i, sem_g, sem_w):
        buf_v = [buf_v0, buf_v1]
        wid = lax.axis_index("s") * _NC + lax.axis_index("c")
        pbase = wid * _PPW
        idx_cp = [
            pltpu.async_copy(
                idx_hbm.at[pl.ds(b * _SEQ + pbase, _PPW)],
                idx_v.at[pl.ds(b * _PPW, _PPW)], sem_i.at[b])
            for b in range(_BATCH)
        ]

        def start_gather(b):
            idx_cp[b].wait()
            return pltpu.async_copy(
                table_hbm.at[idx_v.at[pl.ds(b * _PPW, _CHUNK)]],
                buf_v[b % 2], sem_g.at[b % 2])

        pending_in = start_gather(0)
        pending_out = [None, None]
        for b in range(_BATCH):
            s = b % 2
            if b + 1 < _BATCH:
                if pending_out[1 - s] is not None:
                    pending_out[1 - s].wait()
                    pending_out[1 - s] = None
                nxt = start_gather(b + 1)
            pending_in.wait()
            if b + 1 < _BATCH:
                pending_in = nxt
            pending_out[s] = pltpu.async_copy(
                buf_v[s], out_hbm.at[b, pl.ds(pbase, _CHUNK)], sem_w.at[s])
        for w in pending_out:
            if w is not None:
                w.wait()

    return body(idx_flat, table)


def kernel(x, W):
    idx_flat = x.T.reshape(_BATCH * _SEQ)
    g = _gather(idx_flat, W)
    return g.transpose(1, 0, 2) + jnp.asarray(_PE)
